# Initial kernel scaffold; baseline (speedup 1.0000x reference)
import jax
import jax.numpy as jnp
from jax.experimental import pallas as pl


def kernel(feature_map, keypoints, mask, original_H, original_W):
    raise NotImplementedError("write your pallas kernel here")


# speedup vs baseline: 8.9945x; 8.9945x over previous
import functools

import jax
import jax.numpy as jnp
from jax.experimental import pallas as pl
from jax.experimental.pallas import tpu as pltpu


def _roi_body(fm_ref, cd_ref, sc_ref, out_ref, *, H, W):
    N = sc_ref.shape[2]
    cd = cd_ref[0]
    x0 = cd[0:1, :].reshape(1, 1, N)
    x1 = cd[1:2, :].reshape(1, 1, N)
    y0 = cd[2:3, :].reshape(1, 1, N)
    y1 = cd[3:4, :].reshape(1, 1, N)

    yi = jax.lax.broadcasted_iota(jnp.int32, (H, W, N), 0)
    xi = jax.lax.broadcasted_iota(jnp.int32, (H, W, N), 1)
    inside = (yi >= y0) & (yi < y1) & (xi >= x0) & (xi < x1)
    ind = jnp.where(inside, 1.0, 0.0).astype(jnp.float32).reshape(H * W, N)

    acc = jnp.dot(fm_ref[0], ind, preferred_element_type=jnp.float32)
    out_ref[0] = acc * sc_ref[0]


def kernel(feature_map, keypoints, mask, original_H, original_W):
    B, C, H, W = feature_map.shape
    N = keypoints.shape[1]
    sx = W / original_W
    sy = H / original_H
    x, y, w, h = (keypoints[..., 0], keypoints[..., 1],
                  keypoints[..., 2], keypoints[..., 3])
    xr = jnp.clip((x * sx).astype(jnp.int32), 0, W - 1)
    yr = jnp.clip((y * sy).astype(jnp.int32), 0, H - 1)
    wr = jnp.minimum(jnp.maximum((w * sx).astype(jnp.int32), 1), W - xr)
    hr = jnp.minimum(jnp.maximum((h * sy).astype(jnp.int32), 1), H - yr)
    coords = jnp.stack([xr, xr + wr, yr, yr + hr], axis=1)
    area = (hr * wr).astype(jnp.float32)
    scale = jnp.where(mask > 0, 1.0 / area, 0.0).reshape(B, 1, N)

    fm = feature_map.reshape(B, C, H * W)
    c_blk = 128
    grid = (B, C // c_blk)
    out = pl.pallas_call(
        functools.partial(_roi_body, H=H, W=W),
        grid=grid,
        in_specs=[
            pl.BlockSpec((1, c_blk, H * W), lambda b, c: (b, c, 0)),
            pl.BlockSpec((1, 4, N), lambda b, c: (b, 0, 0)),
            pl.BlockSpec((1, 1, N), lambda b, c: (b, 0, 0)),
        ],
        out_specs=pl.BlockSpec((1, c_blk, N), lambda b, c: (b, c, 0)),
        out_shape=jax.ShapeDtypeStruct((B, C, N), jnp.float32),
        compiler_params=pltpu.CompilerParams(
            dimension_semantics=("parallel", "arbitrary"),
            vmem_limit_bytes=50 * 1024 * 1024,
        ),
    )(fm, coords, scale)
    return jnp.transpose(out, (0, 2, 1))
